# Initial kernel scaffold; baseline (speedup 1.0000x reference)
import functools

import jax
import jax.numpy as jnp
from jax import lax
from jax.experimental import pallas as pl
from jax.experimental.pallas import tpu as pltpu
from jax.experimental.pallas import tpu_sc as plsc

EMBED_DIM = 32
NC = 2
NS = 16
NW = NC * NS
GATHER = 128
SUB = 8
BLOCK = GATHER * SUB


def _body(idx_hbm, table_hbm, out_hbm, idx_v, rows_v, sem):
    wid = lax.axis_index("s") * NC + lax.axis_index("c")
    idx_rows_per_w = idx_hbm.shape[0] // NW
    blocks_per_w = idx_rows_per_w // SUB
    row0 = wid * idx_rows_per_w

    def blk(i, carry):
        r = row0 + i * SUB
        pltpu.sync_copy(idx_hbm.at[pl.ds(r, SUB)], idx_v)
        descs = [
            pltpu.async_copy(
                table_hbm.at[idx_v.at[j]],
                rows_v.at[pl.ds(j * GATHER, GATHER)],
                sem,
            )
            for j in range(SUB)
        ]
        for d in descs:
            d.wait()
        pltpu.sync_copy(rows_v, out_hbm.at[pl.ds(r * GATHER, BLOCK)])
        return carry

    lax.fori_loop(0, blocks_per_w, blk, 0)


@functools.partial(jax.jit, static_argnums=(2,))
def _gather(idx2d, table, n_rows):
    mesh = plsc.VectorSubcoreMesh(core_axis_name="c", subcore_axis_name="s")
    k = pl.kernel(
        _body,
        out_type=jax.ShapeDtypeStruct((n_rows, EMBED_DIM), jnp.float32),
        mesh=mesh,
        scratch_types=[
            pltpu.VMEM((SUB, GATHER), jnp.int32),
            pltpu.VMEM((BLOCK, EMBED_DIM), jnp.float32),
            pltpu.SemaphoreType.DMA,
        ],
    )
    return k(idx2d, table)


def kernel(relation, table):
    b, s = relation.shape
    n_rows = b * s
    idx2d = relation.reshape(n_rows // GATHER, GATHER).astype(jnp.int32)
    out = _gather(idx2d, table, n_rows)
    return out.reshape(b, s, EMBED_DIM)


# speedup vs baseline: 4.8100x; 4.8100x over previous
import functools

import jax
import jax.numpy as jnp
from jax import lax
from jax.experimental import pallas as pl
from jax.experimental.pallas import tpu as pltpu
from jax.experimental.pallas import tpu_sc as plsc

EMBED_DIM = 32
NC = 2
NS = 16
NW = NC * NS
GATHER = 128
SUB = 8
BLOCK = GATHER * SUB


def _body(idx_hbm, table_hbm, out_hbm, idx_v, rows_v, sem):
    wid = lax.axis_index("s") * NC + lax.axis_index("c")
    idx_rows_per_w = idx_hbm.shape[0] // NW
    blocks_per_w = idx_rows_per_w // SUB
    row0 = wid * idx_rows_per_w

    def blk(i, carry):
        r = row0 + i * SUB
        pltpu.sync_copy(idx_hbm.at[pl.ds(r, SUB)], idx_v)
        descs = [
            pltpu.async_copy(
                table_hbm.at[idx_v.at[j]],
                rows_v.at[pl.ds(j * GATHER, GATHER)],
                sem,
            )
            for j in range(SUB)
        ]
        for d in descs:
            d.wait()
        pltpu.sync_copy(rows_v, out_hbm.at[pl.ds(r * GATHER, BLOCK)])
        return carry

    lax.fori_loop(0, blocks_per_w, blk, 0)


@functools.partial(jax.jit, static_argnums=(2,))
def _gather(idx2d, table, n_rows):
    mesh = plsc.VectorSubcoreMesh(core_axis_name="c", subcore_axis_name="s")
    k = pl.kernel(
        _body,
        out_type=jax.ShapeDtypeStruct((n_rows, EMBED_DIM), jnp.float32),
        mesh=mesh,
        scratch_types=[
            pltpu.VMEM((SUB, GATHER), jnp.int32),
            pltpu.VMEM((BLOCK, EMBED_DIM), jnp.float32),
            pltpu.SemaphoreType.DMA,
        ],
        compiler_params=pltpu.CompilerParams(use_tc_tiling_on_sc=False),
    )
    return k(idx2d, table)


def kernel(relation, table):
    b, s = relation.shape
    n_rows = b * s
    idx2d = relation.reshape(n_rows // GATHER, GATHER).astype(jnp.int32)
    out = _gather(idx2d, table, n_rows)
    return out.reshape(b, s, EMBED_DIM)


# speedup vs baseline: 5.0330x; 1.0463x over previous
import functools

import jax
import jax.numpy as jnp
from jax import lax
from jax.experimental import pallas as pl
from jax.experimental.pallas import tpu as pltpu
from jax.experimental.pallas import tpu_sc as plsc

EMBED_DIM = 32
NC = 2
NS = 16
NW = NC * NS
GATHER = 128
SUB = 8
BLOCK = GATHER * SUB


def _body(idx_hbm, table_hbm, out_hbm, idx_v, rows_v, sem_i, sem_g, sem_o):
    wid = lax.axis_index("s") * NC + lax.axis_index("c")
    idx_rows_per_w = idx_hbm.shape[0] // NW
    nblk = idx_rows_per_w // SUB
    row0 = wid * idx_rows_per_w

    def idx_slice(i):
        return idx_hbm.at[pl.ds(row0 + i * SUB, SUB)]

    def out_slice(i):
        return out_hbm.at[pl.ds((row0 + i * SUB) * GATHER, BLOCK)]

    def fire_gathers(i, b):
        for j in range(SUB):
            pltpu.async_copy(
                table_hbm.at[idx_v.at[b, j]],
                rows_v.at[b, pl.ds(j * GATHER, GATHER)],
                sem_g.at[b],
            )

    def drain_gathers(b):
        pltpu.make_async_copy(out_slice(0), rows_v.at[b], sem_g.at[b]).wait()

    def wait_store(i, b):
        pltpu.make_async_copy(rows_v.at[b], out_slice(i), sem_o.at[b]).wait()

    pltpu.sync_copy(idx_slice(0), idx_v.at[0])
    fire_gathers(0, 0)
    pltpu.async_copy(idx_slice(1), idx_v.at[1], sem_i.at[1])

    def step(k, carry):
        i0 = 2 * k
        drain_gathers(0)

        @pl.when(k >= 1)
        def _():
            wait_store(i0 - 1, 1)

        pltpu.make_async_copy(idx_slice(i0 + 1), idx_v.at[1], sem_i.at[1]).wait()
        fire_gathers(i0 + 1, 1)

        @pl.when(i0 + 2 < nblk)
        def _():
            pltpu.async_copy(idx_slice(i0 + 2), idx_v.at[0], sem_i.at[0])

        pltpu.async_copy(rows_v.at[0], out_slice(i0), sem_o.at[0])

        i1 = i0 + 1
        drain_gathers(1)
        wait_store(i1 - 1, 0)

        @pl.when(i1 + 1 < nblk)
        def _():
            pltpu.make_async_copy(idx_slice(i1 + 1), idx_v.at[0], sem_i.at[0]).wait()
            fire_gathers(i1 + 1, 0)
            pltpu.async_copy(idx_slice(i1 + 2), idx_v.at[1], sem_i.at[1])

        pltpu.async_copy(rows_v.at[1], out_slice(i1), sem_o.at[1])
        return carry

    lax.fori_loop(0, nblk // 2, step, 0)

    wait_store(nblk - 1, 1)


@functools.partial(jax.jit, static_argnums=(2,))
def _gather(idx2d, table, n_rows):
    mesh = plsc.VectorSubcoreMesh(core_axis_name="c", subcore_axis_name="s")
    k = pl.kernel(
        _body,
        out_type=jax.ShapeDtypeStruct((n_rows, EMBED_DIM), jnp.float32),
        mesh=mesh,
        scratch_types=[
            pltpu.VMEM((2, SUB, GATHER), jnp.int32),
            pltpu.VMEM((2, BLOCK, EMBED_DIM), jnp.float32),
            pltpu.SemaphoreType.DMA((2,)),
            pltpu.SemaphoreType.DMA((2,)),
            pltpu.SemaphoreType.DMA((2,)),
        ],
        compiler_params=pltpu.CompilerParams(use_tc_tiling_on_sc=False),
    )
    return k(idx2d, table)


def kernel(relation, table):
    b, s = relation.shape
    n_rows = b * s
    idx2d = relation.reshape(n_rows // GATHER, GATHER).astype(jnp.int32)
    out = _gather(idx2d, table, n_rows)
    return out.reshape(b, s, EMBED_DIM)
